# Initial kernel scaffold; baseline (speedup 1.0000x reference)
import jax
import jax.numpy as jnp
from jax.experimental import pallas as pl


def kernel(output, mask, ind, target):
    raise NotImplementedError("write your pallas kernel here")


# speedup vs baseline: 8.3437x; 8.3437x over previous
import jax
import jax.numpy as jnp
import numpy as np
from jax.experimental import pallas as pl
from jax.experimental.pallas import tpu as pltpu

EPS = float(np.finfo(np.float32).eps)
N_R = 18
M_DIR = 384
GATHER_CHUNK = 4096


def _direction_matrix(n):
    rows = []
    for i in range(n):
        for j in range(i + 1, n):
            r = np.zeros(n, np.float32)
            r[i], r[j] = -1.0, 1.0
            rows.append(r)
    for i in range(n):
        for j in range(i + 1, n):
            r = np.zeros(n, np.float32)
            r[i], r[j] = -1.0, -1.0
            rows.append(r)
    for i in range(n):
        r = np.zeros(n, np.float32)
        r[i] = -2.0
        rows.append(r)
    d = np.stack(rows, 0)
    pad = np.zeros((M_DIR - d.shape[0], n), np.float32)
    return np.concatenate([d, pad], 0)


_DMAT_T = _direction_matrix(N_R).T
_ANG = np.linspace(0.0, np.pi, N_R + 1)[:-1]
_TRIG = np.stack([np.cos(_ANG), np.sin(_ANG)]).astype(np.float32)


def _interleave_mats(p):
    e = np.zeros((p, 2 * p), np.float32)
    o = np.zeros((p, 2 * p), np.float32)
    for i in range(p):
        e[i, 2 * i] = 1.0
        o[i, 2 * i + 1] = 1.0
    return e, o


def _dot(a, b):
    return jax.lax.dot_general(a, b, (((1,), (0,)), ((), ())),
                               preferred_element_type=jnp.float32)


def _min_area_rect_corners(r, cosr, sinr, dmat):
    k = r.shape[0]
    x = r * cosr
    y = r * sinr
    dx = _dot(x, dmat)
    dy = _dot(y, dmat)
    nrm = jnp.sqrt(dx * dx + dy * dy)
    nc = jnp.maximum(nrm, 1e-9)
    ux = dx / nc
    uy = dy / nc
    umax = jnp.zeros_like(ux)
    vmax = jnp.zeros_like(ux)
    for s in range(N_R):
        xs = x[:, s:s + 1]
        ys = y[:, s:s + 1]
        umax = jnp.maximum(umax, jnp.abs(xs * ux + ys * uy))
        vmax = jnp.maximum(vmax, jnp.abs(ys * ux - xs * uy))
    area = jnp.where(nrm > 1e-9, (umax + umax) * (vmax + vmax), jnp.inf)
    midx = jnp.argmin(area, axis=-1)[:, None]
    oh = jnp.where(jax.lax.broadcasted_iota(jnp.int32, (k, M_DIR), 1)
                   == midx, 1.0, 0.0)
    uxk = jnp.sum(oh * ux, axis=1, keepdims=True)
    uyk = jnp.sum(oh * uy, axis=1, keepdims=True)
    um = jnp.sum(oh * umax, axis=1, keepdims=True)
    vm = jnp.sum(oh * vmax, axis=1, keepdims=True)
    vxk, vyk = -uyk, uxk
    cx = jnp.concatenate([-um * uxk - vm * vxk, um * uxk - vm * vxk,
                          um * uxk + vm * vxk, -um * uxk + vm * vxk], 1)
    cy = jnp.concatenate([-um * uyk - vm * vyk, um * uyk - vm * vyk,
                          um * uyk + vm * vyk, -um * uyk + vm * vyk], 1)
    return cx, cy


def _shoelace(px, py):
    nx = jnp.concatenate([px[:, 1:], px[:, :1]], 1)
    ny = jnp.concatenate([py[:, 1:], py[:, :1]], 1)
    s = jnp.sum(px * ny - py * nx, axis=1, keepdims=True)
    return 0.5 * jnp.abs(s)


def _clip_halfplane(px, py, e1x, e1y, e2x, e2y, em, om):
    k, p = px.shape
    q = 2 * p
    nx = jnp.concatenate([px[:, 1:], px[:, :1]], 1)
    ny = jnp.concatenate([py[:, 1:], py[:, :1]], 1)
    dex = e2x - e1x
    dey = e2y - e1y
    side = dex * (py - e1y) - dey * (px - e1x)
    siden = jnp.concatenate([side[:, 1:], side[:, :1]], 1)
    denom = side - siden
    t = side / jnp.where(jnp.abs(denom) < 1e-12, 1e-12, denom)
    ix = px + t * (nx - px)
    iy = py + t * (ny - py)
    s_in = jnp.where(side >= 0, 1.0, 0.0)
    e_in = jnp.where(siden >= 0, 1.0, 0.0)
    xr = jnp.abs(s_in - e_in)
    ox = _dot(ix, em) + _dot(nx, om)
    oy = _dot(iy, em) + _dot(ny, om)
    vf = _dot(xr, em) + _dot(e_in, om)
    have = vf
    for sh in (1, 2, 4, 8, 16, 32):
        if sh >= q:
            break
        zpad = jnp.zeros((k, sh), jnp.float32)
        shx = jnp.concatenate([zpad, ox[:, :q - sh]], 1)
        shy = jnp.concatenate([zpad, oy[:, :q - sh]], 1)
        shh = jnp.concatenate([zpad, have[:, :q - sh]], 1)
        keep = have > 0.5
        ox = jnp.where(keep, ox, shx)
        oy = jnp.where(keep, oy, shy)
        have = jnp.maximum(have, shh)
    iota = jax.lax.broadcasted_iota(jnp.int32, (k, q), 1)
    valid = vf > 0.5
    fv = jnp.min(jnp.where(valid, iota, q), axis=1, keepdims=True)
    foh = jnp.where(iota == fv, 1.0, 0.0)
    fvx = jnp.sum(foh * ox, axis=1, keepdims=True)
    fvy = jnp.sum(foh * oy, axis=1, keepdims=True)
    miss = have < 0.5
    ox = jnp.where(miss, fvx, ox)
    oy = jnp.where(miss, fvy, oy)
    anyv = jnp.max(vf, axis=1, keepdims=True) > 0.5
    ox = jnp.where(anyv, ox, 0.0)
    oy = jnp.where(anyv, oy, 0.0)
    return ox, oy


def _batch_kernel(ind_ref, mask_ref, target_ref, trig_ref, dmat_ref,
                  e4_ref, o4_ref, e8_ref, o8_ref, e16_ref, o16_ref,
                  e32_ref, o32_ref, feat_ref, out_ref):
    kk = ind_ref.shape[-1]
    n = feat_ref.shape[1]
    hw = feat_ref.shape[2]
    ind = ind_ref[0, 0, :][:, None]
    ch = GATHER_CHUNK if hw % GATHER_CHUNK == 0 else hw
    pred = jnp.zeros((kk, n), jnp.float32)
    for c in range(hw // ch):
        iota = jax.lax.broadcasted_iota(jnp.int32, (kk, ch), 1) + (c * ch)
        oh = jnp.where(iota == ind, 1.0, 0.0)
        fc = feat_ref[0, :, c * ch:(c + 1) * ch]
        pred = pred + jax.lax.dot_general(
            oh, fc, (((1,), (1,)), ((), ())),
            preferred_element_type=jnp.float32)
    target = target_ref[0]
    mf = mask_ref[0, 0, :][:, None].astype(jnp.float32)
    d = pred - target
    ad = jnp.abs(d)
    sl1 = jnp.sum(jnp.where(ad < 1.0, 0.5 * d * d, ad - 0.5),
                  axis=1, keepdims=True) / float(n)
    cosr = trig_ref[0:1, :]
    sinr = trig_ref[1:2, :]
    dmat = dmat_ref[...]
    ilv = {4: (e4_ref[...], o4_ref[...]), 8: (e8_ref[...], o8_ref[...]),
           16: (e16_ref[...], o16_ref[...]),
           32: (e32_ref[...], o32_ref[...])}
    cax, cay = _min_area_rect_corners(pred, cosr, sinr, dmat)
    cbx, cby = _min_area_rect_corners(target, cosr, sinr, dmat)
    px, py = cax, cay
    for t in range(4):
        t2 = (t + 1) % 4
        em, om = ilv[px.shape[1]]
        px, py = _clip_halfplane(px, py,
                                 cbx[:, t:t + 1], cby[:, t:t + 1],
                                 cbx[:, t2:t2 + 1], cby[:, t2:t2 + 1],
                                 em, om)
    inter = _shoelace(px, py)
    aa = _shoelace(cax, cay)
    ab = _shoelace(cbx, cby)
    union = aa + ab - inter
    iou = inter / jnp.maximum(union, 1e-12)
    alpha = -jnp.log(jnp.abs(iou) + EPS)
    li = alpha * sl1 / (jnp.abs(sl1) + EPS)
    s_sl1 = jnp.sum(sl1 * mf)
    s_li = jnp.sum(li * mf)
    s_mf = jnp.sum(mf)
    lane = jax.lax.broadcasted_iota(jnp.int32, (1, 128), 1)
    vec = (jnp.where(lane == 0, s_sl1, 0.0)
           + jnp.where(lane == 1, s_li, 0.0)
           + jnp.where(lane == 2, s_mf, 0.0))
    out_ref[0] = vec


def _const_spec(shape):
    nd = len(shape)
    return pl.BlockSpec(shape, lambda i, _nd=nd: (0,) * _nd)


@jax.jit
def kernel(output, mask, ind, target):
    b, n, h, w = output.shape
    k = ind.shape[1]
    feat = output.reshape(b, n, h * w)
    ind3 = ind.astype(jnp.int32).reshape(b, 1, k)
    mask3 = mask.reshape(b, 1, k)
    consts = [jnp.asarray(_TRIG), jnp.asarray(_DMAT_T)]
    for p in (4, 8, 16, 32):
        e, o = _interleave_mats(p)
        consts += [jnp.asarray(e), jnp.asarray(o)]
    partial = pl.pallas_call(
        _batch_kernel,
        grid=(b,),
        in_specs=[
            pl.BlockSpec((1, 1, k), lambda i: (i, 0, 0)),
            pl.BlockSpec((1, 1, k), lambda i: (i, 0, 0)),
            pl.BlockSpec((1, k, n), lambda i: (i, 0, 0)),
            _const_spec((2, n)),
            _const_spec((n, M_DIR)),
            _const_spec((4, 8)), _const_spec((4, 8)),
            _const_spec((8, 16)), _const_spec((8, 16)),
            _const_spec((16, 32)), _const_spec((16, 32)),
            _const_spec((32, 64)), _const_spec((32, 64)),
            pl.BlockSpec((1, n, h * w), lambda i: (i, 0, 0)),
        ],
        out_specs=pl.BlockSpec((1, 1, 128), lambda i: (i, 0, 0)),
        out_shape=jax.ShapeDtypeStruct((b, 1, 128), jnp.float32),
        compiler_params=pltpu.CompilerParams(
            dimension_semantics=("parallel",)),
    )(ind3, mask3, target, *consts, feat)
    s = jnp.sum(partial[:, 0, :], axis=0)
    cnt = jnp.maximum(s[2], 1.0)
    return s[0] / cnt, s[1] / cnt
